# baseline (speedup 1.0000x reference)
import functools

import jax
import jax.numpy as jnp
from jax import lax
from jax.experimental import pallas as pl
from jax.experimental.pallas import tpu as pltpu
from jax.experimental.pallas import tpu_sc as plsc

_NUM_CORES = 2
_NUM_SUBCORES = 16
_NUM_WORKERS = _NUM_CORES * _NUM_SUBCORES
_CHUNKS = 4


@jax.jit
def _embed(x, W):
    B, = x.shape
    V, D = W.shape
    b_per_w = B // _NUM_WORKERS
    cpw = b_per_w // _CHUNKS

    mesh = plsc.VectorSubcoreMesh(core_axis_name="c", subcore_axis_name="s")

    @functools.partial(
        pl.kernel,
        mesh=mesh,
        out_type=jax.ShapeDtypeStruct((B, D), jnp.float32),
        scratch_types=[
            pltpu.VMEM((_CHUNKS, cpw), jnp.int32),
            pltpu.VMEM((2, cpw, D), jnp.float32),
            pltpu.SemaphoreType.DMA,
            pltpu.SemaphoreType.DMA,
            pltpu.SemaphoreType.DMA,
            pltpu.SemaphoreType.DMA,
        ],
    )
    def k(x_hbm, w_hbm, out_hbm, idx_v, rows_v, gsem0, gsem1, wsem0, wsem1):
        wid = lax.axis_index("s") * _NUM_CORES + lax.axis_index("c")
        base = wid * b_per_w
        gsems = (gsem0, gsem1)
        wsems = (wsem0, wsem1)

        pltpu.sync_copy(x_hbm.at[pl.ds(base, cpw)], idx_v.at[0])
        gh = [None] * _CHUNKS
        wh = [None] * _CHUNKS
        gh[0] = pltpu.async_copy(w_hbm.at[idx_v.at[0]], rows_v.at[0], gsems[0])
        for c in range(1, _CHUNKS):
            pltpu.sync_copy(
                x_hbm.at[pl.ds(base + c * cpw, cpw)], idx_v.at[c])

        for c in range(_CHUNKS):
            buf = c % 2
            gh[c].wait()
            if c + 1 < _CHUNKS:
                if c >= 1:
                    wh[c - 1].wait()
                gh[c + 1] = pltpu.async_copy(
                    w_hbm.at[idx_v.at[c + 1]], rows_v.at[1 - buf],
                    gsems[1 - buf])
            wh[c] = pltpu.async_copy(
                rows_v.at[buf], out_hbm.at[pl.ds(base + c * cpw, cpw)],
                wsems[buf])
        if _CHUNKS >= 2:
            wh[_CHUNKS - 2].wait()
        wh[_CHUNKS - 1].wait()

    return k(x, W)


def kernel(x, W):
    return _embed(x.astype(jnp.int32), W)


# speedup vs baseline: 1.5804x; 1.5804x over previous
import functools

import jax
import jax.numpy as jnp
from jax import lax
from jax.experimental import pallas as pl
from jax.experimental.pallas import tpu as pltpu
from jax.experimental.pallas import tpu_sc as plsc

_NUM_CORES = 2
_NUM_SUBCORES = 16
_NUM_WORKERS = _NUM_CORES * _NUM_SUBCORES
_CHUNKS = 4


@jax.jit
def _embed(x, W):
    B, = x.shape
    V, D = W.shape
    b_per_w = B // _NUM_WORKERS
    cpw = b_per_w // _CHUNKS
    ngrp = D // 16

    mesh = plsc.VectorSubcoreMesh(core_axis_name="c", subcore_axis_name="s")

    @functools.partial(
        pl.kernel,
        mesh=mesh,
        out_type=jax.ShapeDtypeStruct((B, D), jnp.float32),
        scratch_types=[
            pltpu.VMEM((b_per_w,), jnp.int32),
            pltpu.VMEM((V, D), jnp.float32),
            pltpu.VMEM((2, cpw, D), jnp.float32),
            pltpu.SemaphoreType.DMA,
            pltpu.SemaphoreType.DMA,
        ],
    )
    def k(x_hbm, w_hbm, out_hbm, idx_v, table_v, out_v, wsem0, wsem1):
        wid = lax.axis_index("s") * _NUM_CORES + lax.axis_index("c")
        base = wid * b_per_w
        wsems = (wsem0, wsem1)

        pltpu.sync_copy(w_hbm, table_v)
        pltpu.sync_copy(x_hbm.at[pl.ds(base, b_per_w)], idx_v)

        wh = [None] * _CHUNKS
        for c in range(_CHUNKS):
            buf = c % 2
            if c >= 2:
                wh[c - 2].wait()

            def body(rb, carry, c=c, buf=buf):
                idxv = idx_v[pl.ds(c * cpw + rb * 16, 16)]
                rbase = rb * 16
                for l in range(16):
                    row = idxv[l]
                    for g in range(ngrp):
                        out_v[buf, rbase + l, pl.ds(g * 16, 16)] = (
                            table_v[row, pl.ds(g * 16, 16)])
                return carry

            lax.fori_loop(0, cpw // 16, body, None)
            wh[c] = pltpu.async_copy(
                out_v.at[buf], out_hbm.at[pl.ds(base + c * cpw, cpw)],
                wsems[buf])
        wh[_CHUNKS - 2].wait()
        wh[_CHUNKS - 1].wait()

    return k(x, W)


def kernel(x, W):
    return _embed(x.astype(jnp.int32), W)
